# speedup vs baseline: 6.6451x; 6.6451x over previous
import jax
import jax.numpy as jnp
from jax.experimental import pallas as pl
from jax.experimental.pallas import tpu as pltpu

_B, _N = 16, 8192
_IN, _H, _EMB, _HA, _OUT = 64, 128, 64, 128, 64
_NUM_AISLES = 32


def _lrelu(v):
    return jnp.where(v > 0, v, 0.01 * v)


def _fused_kernel(x_ref, ids_ref, mask_ref,
                  w1_ref, b1_ref, w2_ref, b2_ref, w3_ref, b3_ref,
                  w4a_ref, w4b_ref, b4_ref, w5_ref, b5_ref, w6_ref, b6_ref,
                  out_ref):
    f32 = jnp.float32
    xb = x_ref[...]
    h = _lrelu(jnp.dot(xb, w1_ref[...], preferred_element_type=f32) + b1_ref[...])
    h = _lrelu(jnp.dot(h, w2_ref[...], preferred_element_type=f32) + b2_ref[...])
    z = jnp.dot(h, w3_ref[...], preferred_element_type=f32) + b3_ref[...]

    ids = ids_ref[0]
    oh = (jnp.broadcast_to(ids, (_NUM_AISLES, _N)) ==
          jax.lax.broadcasted_iota(jnp.int32, (_NUM_AISLES, _N), 0)).astype(f32)
    sums = jax.lax.dot_general(oh, z, (((1,), (0,)), ((), ())),
                               preferred_element_type=f32)
    counts = jnp.sum(oh, axis=1, keepdims=True)
    means = sums / jnp.maximum(counts, 1.0)
    g = jax.lax.dot_general(oh, means, (((0,), (0,)), ((), ())),
                            preferred_element_type=f32)

    h2 = _lrelu(jnp.dot(z, w4a_ref[...], preferred_element_type=f32)
                + jnp.dot(g, w4b_ref[...], preferred_element_type=f32)
                + b4_ref[...])
    h2 = _lrelu(jnp.dot(h2, w5_ref[...], preferred_element_type=f32) + b5_ref[...])
    scores = jax.lax.dot_general(w6_ref[...], h2, (((0,), (1,)), ((), ())),
                                 preferred_element_type=f32) + b6_ref[0, 0]

    mk = mask_ref[0]
    logits = jnp.where(mk != 0, scores, -jnp.inf)
    mx = jnp.max(logits, axis=1, keepdims=True)
    e = jnp.exp(logits - mx)
    out_ref[0] = e / jnp.sum(e, axis=1, keepdims=True)


def kernel(x, aisle_nrs, mask, W1, b1, W2, b2, W3, b3, W4, b4, W5, b5, W6, b6):
    ids = aisle_nrs.astype(jnp.int32).reshape(_B, 1, _N)
    mask3 = mask.astype(jnp.int32).reshape(_B, 1, _N)
    W4a, W4b = W4[:_EMB], W4[_EMB:]

    full = lambda arr: pl.BlockSpec(arr.shape, lambda b: (0,) * arr.ndim)
    row2d = pl.BlockSpec((1, 1, _N), lambda b: (b, 0, 0))

    weights = [W1, b1.reshape(1, _H), W2, b2.reshape(1, _H), W3, b3.reshape(1, _EMB),
               W4a, W4b, b4.reshape(1, _HA), W5, b5.reshape(1, _OUT), W6,
               b6.reshape(1, 1)]

    probs = pl.pallas_call(
        _fused_kernel,
        grid=(_B,),
        in_specs=[pl.BlockSpec((_N, _IN), lambda b: (b, 0)),
                  row2d, row2d] + [full(w) for w in weights],
        out_specs=pl.BlockSpec((1, 1, _N), lambda b: (b, 0, 0)),
        out_shape=jax.ShapeDtypeStruct((_B, 1, _N), jnp.float32),
        compiler_params=pltpu.CompilerParams(
            dimension_semantics=("arbitrary",)),
    )(x, ids, mask3, *weights)

    return probs.reshape(_B, _N)


# speedup vs baseline: 7.2656x; 1.0934x over previous
import jax
import jax.numpy as jnp
from jax.experimental import pallas as pl
from jax.experimental.pallas import tpu as pltpu

_B, _N = 16, 8192
_IN, _H, _EMB, _HA, _OUT = 64, 128, 64, 128, 64
_NUM_AISLES = 32


def _lrelu(v):
    return jnp.where(v > 0, v, 0.01 * v)


def _fused_kernel(x_ref, ids_ref, mask_ref,
                  w1_ref, b1_ref, w2_ref, b2_ref, w3_ref, b3_ref,
                  w4_ref, b4_ref, w5_ref, b5_ref, w6_ref, b6_ref,
                  out_ref):
    f32, bf = jnp.float32, jnp.bfloat16
    xb = x_ref[...]
    h = _lrelu(jnp.dot(xb, w1_ref[...], preferred_element_type=f32) + b1_ref[...])
    h = _lrelu(jnp.dot(h.astype(bf), w2_ref[...], preferred_element_type=f32) + b2_ref[...])
    z = jnp.dot(h.astype(bf), w3_ref[...], preferred_element_type=f32) + b3_ref[...]
    zb = z.astype(bf)

    ids = ids_ref[0]
    oh = (jnp.broadcast_to(ids, (_NUM_AISLES, _N)) ==
          jax.lax.broadcasted_iota(jnp.int32, (_NUM_AISLES, _N), 0)).astype(bf)
    sums = jax.lax.dot_general(oh, zb, (((1,), (0,)), ((), ())),
                               preferred_element_type=f32)
    counts = jnp.sum(oh.astype(f32), axis=1, keepdims=True)
    means = (sums / jnp.maximum(counts, 1.0)).astype(bf)
    g = jax.lax.dot_general(oh, means, (((0,), (0,)), ((), ())),
                            preferred_element_type=f32)

    cat = jnp.concatenate([zb, g.astype(bf)], axis=1)
    h2 = _lrelu(jnp.dot(cat, w4_ref[...], preferred_element_type=f32) + b4_ref[...])
    h2 = _lrelu(jnp.dot(h2.astype(bf), w5_ref[...], preferred_element_type=f32) + b5_ref[...])
    scores = jax.lax.dot_general(w6_ref[...], h2.astype(bf), (((0,), (1,)), ((), ())),
                                 preferred_element_type=f32) + b6_ref[0, 0]

    mk = mask_ref[0]
    logits = jnp.where(mk != 0, scores, -jnp.inf)
    mx = jnp.max(logits, axis=1, keepdims=True)
    e = jnp.exp(logits - mx)
    out_ref[0] = e / jnp.sum(e, axis=1, keepdims=True)


def kernel(x, aisle_nrs, mask, W1, b1, W2, b2, W3, b3, W4, b4, W5, b5, W6, b6):
    ids = aisle_nrs.astype(jnp.int32).reshape(_B, 1, _N)
    mask3 = mask.astype(jnp.int32).reshape(_B, 1, _N)
    bf = jnp.bfloat16
    x = x.astype(bf)

    full = lambda arr: pl.BlockSpec(arr.shape, lambda b: (0,) * arr.ndim)
    row2d = pl.BlockSpec((1, 1, _N), lambda b: (b, 0, 0))

    weights = [W1.astype(bf), b1.reshape(1, _H), W2.astype(bf), b2.reshape(1, _H),
               W3.astype(bf), b3.reshape(1, _EMB),
               W4.astype(bf), b4.reshape(1, _HA), W5.astype(bf), b5.reshape(1, _OUT),
               W6.astype(bf), b6.reshape(1, 1)]

    probs = pl.pallas_call(
        _fused_kernel,
        grid=(_B,),
        in_specs=[pl.BlockSpec((_N, _IN), lambda b: (b, 0)),
                  row2d, row2d] + [full(w) for w in weights],
        out_specs=pl.BlockSpec((1, 1, _N), lambda b: (b, 0, 0)),
        out_shape=jax.ShapeDtypeStruct((_B, 1, _N), jnp.float32),
        compiler_params=pltpu.CompilerParams(
            dimension_semantics=("arbitrary",)),
    )(x, ids, mask3, *weights)

    return probs.reshape(_B, _N)
